# Initial kernel scaffold; baseline (speedup 1.0000x reference)
import jax
import jax.numpy as jnp
from jax.experimental import pallas as pl


def kernel(embeddings, mom_embeddings, k, mask, warmup):
    raise NotImplementedError("write your pallas kernel here")


# speedup vs baseline: 28.4782x; 28.4782x over previous
import numpy as np
import jax
import jax.numpy as jnp
from jax.experimental import pallas as pl

_TAU = 0.1
_NUM_NEG = 64
_PROX = 40
_C, _H, _W = 128, 84, 84
_NOFF = 4
_NSHIFT = _NOFF * _NOFF
_PAD = _PROX + _NOFF - 1


def _neg_shift_counts() -> np.ndarray:
    n = _H * _W
    rng = np.random.default_rng(0)
    off_r = rng.integers(_PROX, _H - _PROX, size=(n, _NUM_NEG))
    off_c = rng.integers(_PROX, _W - _PROX, size=(n, _NUM_NEG))
    s = (off_r - _PROX) * _NOFF + (off_c - _PROX)
    cnt = np.zeros((n, _NSHIFT), np.float32)
    np.add.at(cnt, (np.arange(n)[:, None], s), 1.0)
    return np.ascontiguousarray(cnt.T).reshape(_NSHIFT, _H, _W)


_CNT = _neg_shift_counts()


def _loss_kernel(emb_ref, mom_pad_ref, cnt_ref, mask_ref, out_ref):
    inv_tau = 1.0 / _TAU
    pos = jnp.sum(emb_ref[...] * mom_pad_ref[:_H, :_W, :], axis=-1)
    dpos = (1.0 + pos) * inv_tau

    def body(s, s_sum):
        dr = _PROX + s // _NOFF
        dc = _PROX + s % _NOFF
        mom_s = mom_pad_ref[pl.ds(dr, _H), pl.ds(dc, _W), :]
        d = jnp.sum(emb_ref[...] * mom_s, axis=-1)
        return s_sum + cnt_ref[s] * jnp.exp((1.0 + d) * inv_tau)

    s_sum = jax.lax.fori_loop(0, _NSHIFT, body, jnp.exp(dpos))
    m = mask_ref[...]
    n = float(_H * _W)
    loss = (n * jnp.sum(m * (-dpos))
            + jnp.sum(jnp.log(s_sum)) * jnp.sum(m)) / (n * n)
    out_ref[...] = loss[None, None]


def kernel(embeddings, mom_embeddings, k, mask, warmup):
    emb = jnp.transpose(embeddings, (1, 2, 0))
    mom = jnp.transpose(mom_embeddings, (1, 2, 0))
    mom_pad = jnp.pad(mom, ((0, _PAD), (0, _PAD), (0, 0)), mode="wrap")
    out = pl.pallas_call(
        _loss_kernel,
        out_shape=jax.ShapeDtypeStruct((1, 1), jnp.float32),
    )(emb, mom_pad, jnp.asarray(_CNT), mask)
    return out[0, 0]
